# baseline (speedup 1.0000x reference)
import functools

import jax
import jax.numpy as jnp
from jax import lax
from jax.experimental import pallas as pl
from jax.experimental.pallas import tpu as pltpu
from jax.experimental.pallas import tpu_sc as plsc

_B = 16384
_K = 64
_N = 1000000

_TILE_COLS = (_N + 127) // 128
_KROW_STRIDE = _TILE_COLS * 8 * 128
_TILES = (_K // 8) * _TILE_COLS
_IMG_ROWS = _TILES * 8

_info = plsc.get_sparse_core_info()
_NC = _info.num_cores
_NS = _info.num_subcores
_NW = _NC * _NS
_BPW = _B // _NW
_CHUNKS = _BPW // 16
_R = 8
_TPW = _TILES // _NW
_TREM = _TILES - _TPW * _NW

_mesh = plsc.VectorSubcoreMesh(core_axis_name="c", subcore_axis_name="s")


@functools.partial(
    pl.kernel,
    mesh=_mesh,
    out_type=[
        jax.ShapeDtypeStruct((_IMG_ROWS, 128), jnp.float32),
        jax.ShapeDtypeStruct((_IMG_ROWS, 128), jnp.float32),
    ],
    scratch_types=[pltpu.SemaphoreType.DMA, pltpu.SemaphoreType.DMA],
)
def _unpad(gu_t, gi_t, img_u, img_i, sem_u, sem_i):
    wid = lax.axis_index("s") * _NC + lax.axis_index("c")
    t0 = wid * _TPW

    def copy_tile(t, sem_slot_u, sem_slot_i):
        ta = t // _TILE_COLS
        tb = t - ta * _TILE_COLS
        pltpu.async_copy(
            gu_t.at[pl.ds(8 * ta, 8), pl.ds(128 * tb, 128)],
            img_u.at[pl.ds(8 * t, 8), :],
            sem_slot_u,
        )
        pltpu.async_copy(
            gi_t.at[pl.ds(8 * ta, 8), pl.ds(128 * tb, 128)],
            img_i.at[pl.ds(8 * t, 8), :],
            sem_slot_i,
        )

    def drain_tile(t, sem_slot_u, sem_slot_i):
        pltpu.make_async_copy(
            gu_t.at[pl.ds(0, 8), pl.ds(0, 128)], img_u.at[pl.ds(8 * t, 8), :], sem_slot_u
        ).wait()
        pltpu.make_async_copy(
            gi_t.at[pl.ds(0, 8), pl.ds(0, 128)], img_i.at[pl.ds(8 * t, 8), :], sem_slot_i
        ).wait()

    def body(n, _):
        copy_tile(t0 + n, sem_u, sem_i)

        @pl.when(n >= _R)
        def _():
            drain_tile(t0 + n - _R, sem_u, sem_i)

        return 0

    lax.fori_loop(0, _TPW, body, 0)

    def tail(n, _):
        drain_tile(t0 + _TPW - _R + n, sem_u, sem_i)
        return 0

    lax.fori_loop(0, _R, tail, 0)

    @pl.when(wid < _TREM)
    def _():
        t = _NW * _TPW + wid
        copy_tile(t, sem_u, sem_i)
        drain_tile(t, sem_u, sem_i)


@functools.partial(
    pl.kernel,
    mesh=_mesh,
    compiler_params=pltpu.CompilerParams(use_tc_tiling_on_sc=False),
    out_type=[
        jax.ShapeDtypeStruct((_K, _B), jnp.float32),
        jax.ShapeDtypeStruct((_K, _B), jnp.float32),
    ],
    scratch_types=[
        pltpu.VMEM((_BPW,), jnp.int32),
        pltpu.VMEM((_BPW,), jnp.int32),
        pltpu.VMEM((_R, _BPW), jnp.int32),
        pltpu.VMEM((_R, _BPW), jnp.int32),
        pltpu.VMEM((_K, _BPW), jnp.float32),
        pltpu.VMEM((_K, _BPW), jnp.float32),
        pltpu.SemaphoreType.DMA,
        pltpu.SemaphoreType.DMA,
    ],
)
def _gather_flat(
    flat_u,
    flat_i,
    users_hbm,
    items_hbm,
    out_u,
    out_i,
    ucp_v,
    icp_v,
    uw_v,
    iw_v,
    urows_v,
    irows_v,
    sem_u,
    sem_i,
):
    wid = lax.axis_index("s") * _NC + lax.axis_index("c")
    base = wid * _BPW
    pltpu.sync_copy(users_hbm.at[pl.ds(base, _BPW)], ucp_v)
    pltpu.sync_copy(items_hbm.at[pl.ds(base, _BPW)], icp_v)

    def precompute(cp_v):
        def cb(c, _):
            r = cp_v[pl.ds(c * 16, 16)]
            cp_v[pl.ds(c * 16, 16)] = ((r >> 7) << 10) + (r & 127)
            return 0

        lax.fori_loop(0, _CHUNKS, cb, 0)

    precompute(ucp_v)
    precompute(icp_v)

    def fire(k):
        slot = k & (_R - 1)
        bk = (k >> 3) * _KROW_STRIDE + (k & 7) * 128

        def bb(c, _):
            uw_v[slot, pl.ds(c * 16, 16)] = ucp_v[pl.ds(c * 16, 16)] + bk
            iw_v[slot, pl.ds(c * 16, 16)] = icp_v[pl.ds(c * 16, 16)] + bk
            return 0

        lax.fori_loop(0, _CHUNKS, bb, 0)
        pltpu.async_copy(flat_u.at[uw_v.at[slot]], urows_v.at[k], sem_u)
        pltpu.async_copy(flat_i.at[iw_v.at[slot]], irows_v.at[k], sem_i)

    def drain(kk):
        slot = kk & (_R - 1)
        pltpu.make_async_copy(flat_u.at[uw_v.at[slot]], urows_v.at[kk], sem_u).wait()
        pltpu.make_async_copy(flat_i.at[iw_v.at[slot]], irows_v.at[kk], sem_i).wait()

    def body(k, _):
        fire(k)

        @pl.when(k >= _R)
        def _():
            drain(k - _R)

        return 0

    lax.fori_loop(0, _K, body, 0)

    def tail(t, _):
        drain(_K - _R + t)
        return 0

    lax.fori_loop(0, _R, tail, 0)

    pltpu.sync_copy(urows_v, out_u.at[:, pl.ds(base, _BPW)])
    pltpu.sync_copy(irows_v, out_i.at[:, pl.ds(base, _BPW)])


def kernel(Gu, Gi, users, items):
    img_u, img_i = _unpad(Gu.T, Gi.T)
    flat_u = img_u.reshape(_IMG_ROWS * 128)
    flat_i = img_i.reshape(_IMG_ROWS * 128)
    out_tu, out_ti = _gather_flat(
        flat_u, flat_i, users.astype(jnp.int32), items.astype(jnp.int32)
    )
    return (out_tu.T, out_ti.T)
